# baseline (speedup 1.0000x reference)
import functools

import numpy as np
import jax
import jax.numpy as jnp
from jax import lax
from jax.experimental import pallas as pl
from jax.experimental.pallas import tpu as pltpu
from jax.experimental.pallas import tpu_sc as plsc

_NUM_LEVELS = 16
_BASE_RES = 16
_PER_LEVEL_SCALE = 2.0
_LOG2_HASHMAP = 19
_NUM_FREQS = 6
_N = 262144
_OUT_DIM = 2 * _NUM_LEVELS + 3 * (1 + 2 * _NUM_FREQS)
_HASH_DIM = 2 * _NUM_LEVELS
_NG = 9
_NPB = _N // 128


def _level_meta():
    hashmap = 2 ** _LOG2_HASHMAP
    offsets = [0]
    resolutions = []
    for l in range(_NUM_LEVELS):
        res = int(np.ceil(_BASE_RES * (_PER_LEVEL_SCALE ** l)))
        resolutions.append(res)
        params = min(hashmap, (res + 1) ** 3)
        params = int(np.ceil(params / 8) * 8)
        offsets.append(offsets[-1] + params)
    return offsets, resolutions


_OFFSETS, _RES = _level_meta()
_P1 = int(np.uint32(2654435761).astype(np.int64) - 2 ** 32)
_P2 = 805459861
_MASK = 2 ** _LOG2_HASHMAP - 1

_NW = 32
_PW = _N // _NW
_C = 512
_NCH = _PW // _C
_NB = 8 * _C // 128


def _hash_body(x_hbm, y_hbm, z_hbm, tp_hbm, out_hbm,
               x_s, y_s, z_s, idx_a, idx_b, idx_c, w_a, w_b, w_c,
               fp_a, fp_b, fp_c, out_s, sem, sem_a, sem_b, sem_c):
    wid = lax.axis_index("s") * 2 + lax.axis_index("c")
    iota = jnp.arange(16, dtype=jnp.int32)
    slots = ((idx_a, w_a, fp_a, sem_a), (idx_b, w_b, fp_b, sem_b),
             (idx_c, w_c, fp_c, sem_c))

    def level_idx(l, idx_s, w_s):
        res = _RES[l]
        off = _OFFSETS[l]
        n_params = _OFFSETS[l + 1] - _OFFSETS[l]
        hashed = (res + 1) ** 3 > n_params
        res_f = float(res)

        def idx_body(j, c2):
            s = j * 16
            xf = x_s[pl.ds(s, 16)] * res_f
            yf = y_s[pl.ds(s, 16)] * res_f
            zf = z_s[pl.ds(s, 16)] * res_f
            xi = xf.astype(jnp.int32)
            yi = yf.astype(jnp.int32)
            zi = zf.astype(jnp.int32)
            fx = xf - xi.astype(jnp.float32)
            fy = yf - yi.astype(jnp.float32)
            fz = zf - zi.astype(jnp.float32)
            gx = 1.0 - fx
            gy = 1.0 - fy
            gz = 1.0 - fz
            if hashed:
                hy0 = yi * _P1
                hy1 = hy0 + _P1
                hz0 = zi * _P2
                hz1 = hz0 + _P2
            else:
                r1 = res + 1
                sy0 = yi * r1
                sy1 = sy0 + r1
                sz0 = zi * (r1 * r1)
                sz1 = sz0 + r1 * r1
            jr = j // 8
            jc = (j % 8) * 16
            for c in range(8):
                bx, by, bz = c & 1, (c >> 1) & 1, (c >> 2) & 1
                if hashed:
                    h = (xi + bx) ^ (hy1 if by else hy0) ^ (hz1 if bz else hz0)
                    idx = (h & _MASK) + off
                else:
                    idx = ((xi + bx) + (sy1 if by else sy0)
                           + (sz1 if bz else sz0) + off)
                w = ((fx if bx else gx) * (fy if by else gy)) * (fz if bz else gz)
                idx_s[4 * c + jr, pl.ds(jc, 16)] = idx
                w_s[pl.ds(c * _C + s, 16)] = w
            return c2
        lax.fori_loop(0, _C // 16, idx_body, 0)

    def level_fire(idx_s, fp_s, sem_x):
        def fire_body(j, c2):
            pltpu.async_copy(tp_hbm.at[idx_s.at[j]], fp_s.at[j], sem_x)
            return c2
        lax.fori_loop(0, _NB, fire_body, 0)

    def level_drain(idx_s, fp_s, sem_x):
        def drain_body(j, c2):
            pltpu.make_async_copy(
                tp_hbm.at[idx_s.at[j]], fp_s.at[j], sem_x).wait()
            return c2
        lax.fori_loop(0, _NB, drain_body, 0)

    def level_acc(l, w_s, fp_s):
        ga0, ra0 = (2 * l) // 8, (2 * l) % 8
        ga1, ra1 = (2 * l + 1) // 8, (2 * l + 1) % 8

        def acc_body(j, c2):
            s = j * 16
            jr = j // 8
            jc = (j % 8) * 16
            a0 = jnp.zeros((16,), jnp.float32)
            a1 = jnp.zeros((16,), jnp.float32)
            for c in range(8):
                w = w_s[pl.ds(c * _C + s, 16)]
                v = fp_s[4 * c + jr, pl.ds(jc, 16)]
                f0 = plsc.bitcast(v << 16, jnp.float32)
                f1 = plsc.bitcast(v & jnp.int32(-65536), jnp.float32)
                a0 = a0 + w * f0
                a1 = a1 + w * f1
            pb = j // 8
            poff = (j % 8) * 16
            out_s[pl.ds(ga0 * 4096 + pb * 1024 + ra0 * 128 + poff, 16)] = a0
            out_s[pl.ds(ga1 * 4096 + pb * 1024 + ra1 * 128 + poff, 16)] = a1
            return c2
        lax.fori_loop(0, _C // 16, acc_body, 0)

    def chunk_body(g, carry):
        base = wid * _PW + g * _C
        pltpu.sync_copy(x_hbm.at[pl.ds(base, _C)], x_s)
        pltpu.sync_copy(y_hbm.at[pl.ds(base, _C)], y_s)
        pltpu.sync_copy(z_hbm.at[pl.ds(base, _C)], z_s)

        for l in range(3):
            idx_s, w_s, fp_s, sem_x = slots[l % 3]
            level_idx(l, idx_s, w_s)
            level_fire(idx_s, fp_s, sem_x)
        for l in range(_NUM_LEVELS):
            idx_s, w_s, fp_s, sem_x = slots[l % 3]
            level_drain(idx_s, fp_s, sem_x)
            level_acc(l, w_s, fp_s)
            if l + 3 < _NUM_LEVELS:
                level_idx(l + 3, idx_s, w_s)
                level_fire(idx_s, fp_s, sem_x)

        pbg = base // 128
        for fg in range(4):
            for pb in range(4):
                pltpu.async_copy(
                    out_s.at[pl.ds(fg * 4096 + pb * 1024, 1024)],
                    out_hbm.at[pl.ds((fg * _NPB + pbg + pb) * 1024, 1024)],
                    sem)
        for fg in range(4):
            for pb in range(4):
                pltpu.make_async_copy(
                    out_s.at[pl.ds(fg * 4096 + pb * 1024, 1024)],
                    out_hbm.at[pl.ds((fg * _NPB + pbg + pb) * 1024, 1024)],
                    sem).wait()
        return carry

    lax.fori_loop(0, _NCH, chunk_body, 0)


_hash_call = functools.partial(
    pl.kernel,
    mesh=plsc.VectorSubcoreMesh(core_axis_name="c", subcore_axis_name="s"),
    compiler_params=pltpu.CompilerParams(
        needs_layout_passes=False, use_tc_tiling_on_sc=False),
    out_type=jax.ShapeDtypeStruct((_NG * _NPB * 8 * 128,), jnp.float32),
    scratch_types=[
        pltpu.VMEM((_C,), jnp.float32),
        pltpu.VMEM((_C,), jnp.float32),
        pltpu.VMEM((_C,), jnp.float32),
        pltpu.VMEM((_NB, 128), jnp.int32),
        pltpu.VMEM((_NB, 128), jnp.int32),
        pltpu.VMEM((_NB, 128), jnp.int32),
        pltpu.VMEM((8 * _C,), jnp.float32),
        pltpu.VMEM((8 * _C,), jnp.float32),
        pltpu.VMEM((8 * _C,), jnp.float32),
        pltpu.VMEM((_NB, 128), jnp.int32),
        pltpu.VMEM((_NB, 128), jnp.int32),
        pltpu.VMEM((_NB, 128), jnp.int32),
        pltpu.VMEM((4 * 4 * 8 * 128,), jnp.float32),
        pltpu.SemaphoreType.DMA,
        pltpu.SemaphoreType.DMA,
        pltpu.SemaphoreType.DMA,
        pltpu.SemaphoreType.DMA,
    ],
)(_hash_body)


_PBG = 32


_PBG = 32


def _pe_body(x_ref, y_ref, z_ref, alias_ref, out_ref,
             sx_ref, sy_ref, sz_ref, cx_ref, cy_ref, cz_ref):
    del alias_ref
    g = pl.program_id(1)
    coords = (x_ref[...], y_ref[...], z_ref[...])

    def emit(rows):
        full = jnp.stack(rows, axis=0)
        out_ref[0] = jnp.transpose(full, (1, 0, 2))

    def load_state():
        return ((sx_ref[...], sy_ref[...], sz_ref[...]),
                (cx_ref[...], cy_ref[...], cz_ref[...]))

    def store_state(sin3, cos3):
        sx_ref[...], sy_ref[...], sz_ref[...] = sin3
        cx_ref[...], cy_ref[...], cz_ref[...] = cos3

    def double(sin3, cos3):
        s2 = tuple(2.0 * sv * cv for sv, cv in zip(sin3, cos3))
        c2 = tuple(1.0 - 2.0 * sv * sv for sv in sin3)
        return s2, c2

    @pl.when(g == 0)
    def _():
        sin3 = tuple(jnp.sin(c) for c in coords)
        cos3 = tuple(jnp.cos(c) for c in coords)
        store_state(sin3, cos3)
        emit([coords[0], coords[1], coords[2],
              sin3[0], sin3[1], sin3[2], cos3[0], cos3[1]])

    @pl.when(g == 1)
    def _():
        s0, c0 = load_state()
        s1, c1 = double(s0, c0)
        s2, c2 = double(s1, c1)
        store_state(s2, c2)
        emit([c0[2], s1[0], s1[1], s1[2], c1[0], c1[1], c1[2], s2[0]])

    @pl.when(g == 2)
    def _():
        s2, c2 = load_state()
        s3, c3 = double(s2, c2)
        store_state(s3, c3)
        emit([s2[1], s2[2], c2[0], c2[1], c2[2], s3[0], s3[1], s3[2]])

    @pl.when(g == 3)
    def _():
        s3, c3 = load_state()
        s4, c4 = double(s3, c3)
        store_state(s4, c4)
        emit([c3[0], c3[1], c3[2], s4[0], s4[1], s4[2], c4[0], c4[1]])

    @pl.when(g == 4)
    def _():
        s4, c4 = load_state()
        s5, c5 = double(s4, c4)
        emit([c4[2], s5[0], s5[1], s5[2], c5[0], c5[1], c5[2],
              jnp.zeros_like(coords[0])])


_pe_grid = (_NPB // _PBG, 5)

_pe_call = pl.pallas_call(
    _pe_body,
    grid=_pe_grid,
    in_specs=[
        pl.BlockSpec((_PBG, 128), lambda i, g: (i, 0)),
        pl.BlockSpec((_PBG, 128), lambda i, g: (i, 0)),
        pl.BlockSpec((_PBG, 128), lambda i, g: (i, 0)),
        pl.BlockSpec(memory_space=pl.ANY),
    ],
    out_specs=pl.BlockSpec((1, _PBG, 8, 128), lambda i, g: (4 + g, i, 0, 0)),
    out_shape=jax.ShapeDtypeStruct((_NG, _NPB, 8, 128), jnp.float32),
    input_output_aliases={3: 0},
    scratch_shapes=[pltpu.VMEM((_PBG, 128), jnp.float32) for _ in range(6)],
)


def kernel(position, table):
    xs = position[:, 0]
    ys = position[:, 1]
    zs = position[:, 2]
    b0 = jax.lax.bitcast_convert_type(
        table[:, 0].astype(jnp.bfloat16), jnp.uint16).astype(jnp.int32)
    b1 = jax.lax.bitcast_convert_type(
        table[:, 1].astype(jnp.bfloat16), jnp.uint16).astype(jnp.int32)
    tp = b0 | (b1 << 16)
    flat = _hash_call(xs, ys, zs, tp)
    x2 = xs.reshape(_NPB, 128)
    y2 = ys.reshape(_NPB, 128)
    z2 = zs.reshape(_NPB, 128)
    out4 = _pe_call(x2, y2, z2, flat.reshape(_NG, _NPB, 8, 128))
    return out4.transpose(1, 3, 0, 2).reshape(_N, _NG * 8)[:, :_OUT_DIM]


# speedup vs baseline: 1.0021x; 1.0021x over previous
import functools

import numpy as np
import jax
import jax.numpy as jnp
from jax import lax
from jax.experimental import pallas as pl
from jax.experimental.pallas import tpu as pltpu
from jax.experimental.pallas import tpu_sc as plsc

_NUM_LEVELS = 16
_BASE_RES = 16
_PER_LEVEL_SCALE = 2.0
_LOG2_HASHMAP = 19
_NUM_FREQS = 6
_N = 262144
_OUT_DIM = 2 * _NUM_LEVELS + 3 * (1 + 2 * _NUM_FREQS)
_HASH_DIM = 2 * _NUM_LEVELS
_NG = 9
_NPB = _N // 128


def _level_meta():
    hashmap = 2 ** _LOG2_HASHMAP
    offsets = [0]
    resolutions = []
    for l in range(_NUM_LEVELS):
        res = int(np.ceil(_BASE_RES * (_PER_LEVEL_SCALE ** l)))
        resolutions.append(res)
        params = min(hashmap, (res + 1) ** 3)
        params = int(np.ceil(params / 8) * 8)
        offsets.append(offsets[-1] + params)
    return offsets, resolutions


_OFFSETS, _RES = _level_meta()
_P1 = int(np.uint32(2654435761).astype(np.int64) - 2 ** 32)
_P2 = 805459861
_MASK = 2 ** _LOG2_HASHMAP - 1

_NW = 32
_PW = _N // _NW
_C = 512
_NCH = _PW // _C
_NB = 8 * _C // 128


def _hash_body(x_hbm, y_hbm, z_hbm, tp_hbm, out_hbm,
               x_s, y_s, z_s, idx_a, idx_b, idx_c, w_a, w_b, w_c,
               fp_a, fp_b, fp_c, out_s, sem, sem_a, sem_b, sem_c):
    wid = lax.axis_index("s") * 2 + lax.axis_index("c")
    iota = jnp.arange(16, dtype=jnp.int32)
    slots = ((idx_a, w_a, fp_a, sem_a), (idx_b, w_b, fp_b, sem_b),
             (idx_c, w_c, fp_c, sem_c))

    def level_idx(l, idx_s, w_s):
        res = _RES[l]
        off = _OFFSETS[l]
        n_params = _OFFSETS[l + 1] - _OFFSETS[l]
        hashed = (res + 1) ** 3 > n_params
        res_f = float(res)

        def idx_body(j):
            s = j * 16
            xf = x_s[pl.ds(s, 16)] * res_f
            yf = y_s[pl.ds(s, 16)] * res_f
            zf = z_s[pl.ds(s, 16)] * res_f
            xi = xf.astype(jnp.int32)
            yi = yf.astype(jnp.int32)
            zi = zf.astype(jnp.int32)
            fx = xf - xi.astype(jnp.float32)
            fy = yf - yi.astype(jnp.float32)
            fz = zf - zi.astype(jnp.float32)
            gx = 1.0 - fx
            gy = 1.0 - fy
            gz = 1.0 - fz
            if hashed:
                hy0 = yi * _P1
                hy1 = hy0 + _P1
                hz0 = zi * _P2
                hz1 = hz0 + _P2
            else:
                r1 = res + 1
                sy0 = yi * r1
                sy1 = sy0 + r1
                sz0 = zi * (r1 * r1)
                sz1 = sz0 + r1 * r1
            jr = j // 8
            jc = (j % 8) * 16
            for c in range(8):
                bx, by, bz = c & 1, (c >> 1) & 1, (c >> 2) & 1
                if hashed:
                    h = (xi + bx) ^ (hy1 if by else hy0) ^ (hz1 if bz else hz0)
                    idx = (h & _MASK) + off
                else:
                    idx = ((xi + bx) + (sy1 if by else sy0)
                           + (sz1 if bz else sz0) + off)
                w = ((fx if bx else gx) * (fy if by else gy)) * (fz if bz else gz)
                idx_s[4 * c + jr, pl.ds(jc, 16)] = idx
                w_s[pl.ds(c * _C + s, 16)] = w
        plsc.parallel_loop(0, _C // 16, unroll=2)(idx_body)

    def level_fire(idx_s, fp_s, sem_x):
        def fire_body(j):
            pltpu.async_copy(tp_hbm.at[idx_s.at[j]], fp_s.at[j], sem_x)
        plsc.parallel_loop(0, _NB, unroll=2)(fire_body)

    def level_drain(idx_s, fp_s, sem_x):
        def drain_body(j, c2):
            pltpu.make_async_copy(
                tp_hbm.at[idx_s.at[j]], fp_s.at[j], sem_x).wait()
            return c2
        lax.fori_loop(0, _NB, drain_body, 0)

    def level_acc(l, w_s, fp_s):
        ga0, ra0 = (2 * l) // 8, (2 * l) % 8
        ga1, ra1 = (2 * l + 1) // 8, (2 * l + 1) % 8

        def acc_body(j):
            s = j * 16
            jr = j // 8
            jc = (j % 8) * 16
            a0 = jnp.zeros((16,), jnp.float32)
            a1 = jnp.zeros((16,), jnp.float32)
            for c in range(8):
                w = w_s[pl.ds(c * _C + s, 16)]
                v = fp_s[4 * c + jr, pl.ds(jc, 16)]
                f0 = plsc.bitcast(v << 16, jnp.float32)
                f1 = plsc.bitcast(v & jnp.int32(-65536), jnp.float32)
                a0 = a0 + w * f0
                a1 = a1 + w * f1
            pb = j // 8
            poff = (j % 8) * 16
            out_s[pl.ds(ga0 * 4096 + pb * 1024 + ra0 * 128 + poff, 16)] = a0
            out_s[pl.ds(ga1 * 4096 + pb * 1024 + ra1 * 128 + poff, 16)] = a1
        plsc.parallel_loop(0, _C // 16, unroll=2)(acc_body)

    def chunk_body(g, carry):
        base = wid * _PW + g * _C
        pltpu.sync_copy(x_hbm.at[pl.ds(base, _C)], x_s)
        pltpu.sync_copy(y_hbm.at[pl.ds(base, _C)], y_s)
        pltpu.sync_copy(z_hbm.at[pl.ds(base, _C)], z_s)

        for l in range(3):
            idx_s, w_s, fp_s, sem_x = slots[l % 3]
            level_idx(l, idx_s, w_s)
            level_fire(idx_s, fp_s, sem_x)
        for l in range(_NUM_LEVELS):
            idx_s, w_s, fp_s, sem_x = slots[l % 3]
            level_drain(idx_s, fp_s, sem_x)
            level_acc(l, w_s, fp_s)
            if l + 3 < _NUM_LEVELS:
                level_idx(l + 3, idx_s, w_s)
                level_fire(idx_s, fp_s, sem_x)

        pbg = base // 128
        for fg in range(4):
            for pb in range(4):
                pltpu.async_copy(
                    out_s.at[pl.ds(fg * 4096 + pb * 1024, 1024)],
                    out_hbm.at[pl.ds((fg * _NPB + pbg + pb) * 1024, 1024)],
                    sem)
        for fg in range(4):
            for pb in range(4):
                pltpu.make_async_copy(
                    out_s.at[pl.ds(fg * 4096 + pb * 1024, 1024)],
                    out_hbm.at[pl.ds((fg * _NPB + pbg + pb) * 1024, 1024)],
                    sem).wait()
        return carry

    lax.fori_loop(0, _NCH, chunk_body, 0)


_hash_call = functools.partial(
    pl.kernel,
    mesh=plsc.VectorSubcoreMesh(core_axis_name="c", subcore_axis_name="s"),
    compiler_params=pltpu.CompilerParams(
        needs_layout_passes=False, use_tc_tiling_on_sc=False),
    out_type=jax.ShapeDtypeStruct((_NG * _NPB * 8 * 128,), jnp.float32),
    scratch_types=[
        pltpu.VMEM((_C,), jnp.float32),
        pltpu.VMEM((_C,), jnp.float32),
        pltpu.VMEM((_C,), jnp.float32),
        pltpu.VMEM((_NB, 128), jnp.int32),
        pltpu.VMEM((_NB, 128), jnp.int32),
        pltpu.VMEM((_NB, 128), jnp.int32),
        pltpu.VMEM((8 * _C,), jnp.float32),
        pltpu.VMEM((8 * _C,), jnp.float32),
        pltpu.VMEM((8 * _C,), jnp.float32),
        pltpu.VMEM((_NB, 128), jnp.int32),
        pltpu.VMEM((_NB, 128), jnp.int32),
        pltpu.VMEM((_NB, 128), jnp.int32),
        pltpu.VMEM((4 * 4 * 8 * 128,), jnp.float32),
        pltpu.SemaphoreType.DMA,
        pltpu.SemaphoreType.DMA,
        pltpu.SemaphoreType.DMA,
        pltpu.SemaphoreType.DMA,
    ],
)(_hash_body)


_PBG = 32


_PBG = 32


def _pe_body(x_ref, y_ref, z_ref, alias_ref, out_ref,
             sx_ref, sy_ref, sz_ref, cx_ref, cy_ref, cz_ref):
    del alias_ref
    g = pl.program_id(1)
    coords = (x_ref[...], y_ref[...], z_ref[...])

    def emit(rows):
        full = jnp.stack(rows, axis=0)
        out_ref[0] = jnp.transpose(full, (1, 0, 2))

    def load_state():
        return ((sx_ref[...], sy_ref[...], sz_ref[...]),
                (cx_ref[...], cy_ref[...], cz_ref[...]))

    def store_state(sin3, cos3):
        sx_ref[...], sy_ref[...], sz_ref[...] = sin3
        cx_ref[...], cy_ref[...], cz_ref[...] = cos3

    def double(sin3, cos3):
        s2 = tuple(2.0 * sv * cv for sv, cv in zip(sin3, cos3))
        c2 = tuple(1.0 - 2.0 * sv * sv for sv in sin3)
        return s2, c2

    @pl.when(g == 0)
    def _():
        sin3 = tuple(jnp.sin(c) for c in coords)
        cos3 = tuple(jnp.cos(c) for c in coords)
        store_state(sin3, cos3)
        emit([coords[0], coords[1], coords[2],
              sin3[0], sin3[1], sin3[2], cos3[0], cos3[1]])

    @pl.when(g == 1)
    def _():
        s0, c0 = load_state()
        s1, c1 = double(s0, c0)
        s2, c2 = double(s1, c1)
        store_state(s2, c2)
        emit([c0[2], s1[0], s1[1], s1[2], c1[0], c1[1], c1[2], s2[0]])

    @pl.when(g == 2)
    def _():
        s2, c2 = load_state()
        s3, c3 = double(s2, c2)
        store_state(s3, c3)
        emit([s2[1], s2[2], c2[0], c2[1], c2[2], s3[0], s3[1], s3[2]])

    @pl.when(g == 3)
    def _():
        s3, c3 = load_state()
        s4, c4 = double(s3, c3)
        store_state(s4, c4)
        emit([c3[0], c3[1], c3[2], s4[0], s4[1], s4[2], c4[0], c4[1]])

    @pl.when(g == 4)
    def _():
        s4, c4 = load_state()
        s5, c5 = double(s4, c4)
        emit([c4[2], s5[0], s5[1], s5[2], c5[0], c5[1], c5[2],
              jnp.zeros_like(coords[0])])


_pe_grid = (_NPB // _PBG, 5)

_pe_call = pl.pallas_call(
    _pe_body,
    grid=_pe_grid,
    in_specs=[
        pl.BlockSpec((_PBG, 128), lambda i, g: (i, 0)),
        pl.BlockSpec((_PBG, 128), lambda i, g: (i, 0)),
        pl.BlockSpec((_PBG, 128), lambda i, g: (i, 0)),
        pl.BlockSpec(memory_space=pl.ANY),
    ],
    out_specs=pl.BlockSpec((1, _PBG, 8, 128), lambda i, g: (4 + g, i, 0, 0)),
    out_shape=jax.ShapeDtypeStruct((_NG, _NPB, 8, 128), jnp.float32),
    input_output_aliases={3: 0},
    scratch_shapes=[pltpu.VMEM((_PBG, 128), jnp.float32) for _ in range(6)],
)


def kernel(position, table):
    xs = position[:, 0]
    ys = position[:, 1]
    zs = position[:, 2]
    b0 = jax.lax.bitcast_convert_type(
        table[:, 0].astype(jnp.bfloat16), jnp.uint16).astype(jnp.int32)
    b1 = jax.lax.bitcast_convert_type(
        table[:, 1].astype(jnp.bfloat16), jnp.uint16).astype(jnp.int32)
    tp = b0 | (b1 << 16)
    flat = _hash_call(xs, ys, zs, tp)
    x2 = xs.reshape(_NPB, 128)
    y2 = ys.reshape(_NPB, 128)
    z2 = zs.reshape(_NPB, 128)
    out4 = _pe_call(x2, y2, z2, flat.reshape(_NG, _NPB, 8, 128))
    return out4.transpose(1, 3, 0, 2).reshape(_N, _NG * 8)[:, :_OUT_DIM]


# speedup vs baseline: 1.2000x; 1.1975x over previous
import functools

import numpy as np
import jax
import jax.numpy as jnp
from jax import lax
from jax.experimental import pallas as pl
from jax.experimental.pallas import tpu as pltpu
from jax.experimental.pallas import tpu_sc as plsc

_NUM_LEVELS = 16
_BASE_RES = 16
_PER_LEVEL_SCALE = 2.0
_LOG2_HASHMAP = 19
_NUM_FREQS = 6
_N = 262144
_OUT_DIM = 2 * _NUM_LEVELS + 3 * (1 + 2 * _NUM_FREQS)
_HASH_DIM = 2 * _NUM_LEVELS
_NG = 9
_NPB = _N // 128


def _level_meta():
    hashmap = 2 ** _LOG2_HASHMAP
    offsets = [0]
    resolutions = []
    for l in range(_NUM_LEVELS):
        res = int(np.ceil(_BASE_RES * (_PER_LEVEL_SCALE ** l)))
        resolutions.append(res)
        params = min(hashmap, (res + 1) ** 3)
        params = int(np.ceil(params / 8) * 8)
        offsets.append(offsets[-1] + params)
    return offsets, resolutions


_OFFSETS, _RES = _level_meta()
_P1 = int(np.uint32(2654435761).astype(np.int64) - 2 ** 32)
_P2 = 805459861
_MASK = 2 ** _LOG2_HASHMAP - 1

_NW = 32
_PW = _N // _NW
_C = 512
_NCH = _PW // _C
_NB = 8 * _C // 128
_T01 = _OFFSETS[2]


def _hash_body(x_hbm, y_hbm, z_hbm, tp_hbm, out_hbm,
               x_s, y_s, z_s, idx_a, idx_b, idx_c, idx_d, w_a, w_b, w_c, w_d,
               fp_a, fp_b, fp_c, out_s, tbl01_s, sem, sem_a, sem_b, sem_c):
    wid = lax.axis_index("s") * 2 + lax.axis_index("c")
    iota = jnp.arange(16, dtype=jnp.int32)
    slots = ((idx_a, w_a, fp_a, sem_a), (idx_b, w_b, fp_b, sem_b),
             (idx_c, w_c, fp_c, sem_c))

    def level_idx(l, idx_s, w_s):
        res = _RES[l]
        off = _OFFSETS[l]
        n_params = _OFFSETS[l + 1] - _OFFSETS[l]
        hashed = (res + 1) ** 3 > n_params
        res_f = float(res)

        def idx_body(j):
            s = j * 16
            xf = x_s[pl.ds(s, 16)] * res_f
            yf = y_s[pl.ds(s, 16)] * res_f
            zf = z_s[pl.ds(s, 16)] * res_f
            xi = xf.astype(jnp.int32)
            yi = yf.astype(jnp.int32)
            zi = zf.astype(jnp.int32)
            fx = xf - xi.astype(jnp.float32)
            fy = yf - yi.astype(jnp.float32)
            fz = zf - zi.astype(jnp.float32)
            gx = 1.0 - fx
            gy = 1.0 - fy
            gz = 1.0 - fz
            if hashed:
                hy0 = yi * _P1
                hy1 = hy0 + _P1
                hz0 = zi * _P2
                hz1 = hz0 + _P2
            else:
                r1 = res + 1
                sy0 = yi * r1
                sy1 = sy0 + r1
                sz0 = zi * (r1 * r1)
                sz1 = sz0 + r1 * r1
            jr = j // 8
            jc = (j % 8) * 16
            for c in range(8):
                bx, by, bz = c & 1, (c >> 1) & 1, (c >> 2) & 1
                if hashed:
                    h = (xi + bx) ^ (hy1 if by else hy0) ^ (hz1 if bz else hz0)
                    idx = (h & _MASK) + off
                else:
                    idx = ((xi + bx) + (sy1 if by else sy0)
                           + (sz1 if bz else sz0) + off)
                w = ((fx if bx else gx) * (fy if by else gy)) * (fz if bz else gz)
                idx_s[4 * c + jr, pl.ds(jc, 16)] = idx
                w_s[pl.ds(c * _C + s, 16)] = w
        plsc.parallel_loop(0, _C // 16, unroll=2)(idx_body)

    def level_fire(idx_s, fp_s, sem_x):
        def fire_body(j):
            pltpu.async_copy(tp_hbm.at[idx_s.at[j]], fp_s.at[j], sem_x)
        plsc.parallel_loop(0, _NB, unroll=2)(fire_body)

    def level_drain(idx_s, fp_s, sem_x):
        def drain_body(j, c2):
            pltpu.make_async_copy(
                tp_hbm.at[idx_s.at[j]], fp_s.at[j], sem_x).wait()
            return c2
        lax.fori_loop(0, _NB, drain_body, 0)

    def level_acc_local(l, idx_s, w_s):
        ga0, ra0 = (2 * l) // 8, (2 * l) % 8
        ga1, ra1 = (2 * l + 1) // 8, (2 * l + 1) % 8

        def acc_body(j):
            s = j * 16
            jr = j // 8
            jc = (j % 8) * 16
            a0 = jnp.zeros((16,), jnp.float32)
            a1 = jnp.zeros((16,), jnp.float32)
            for c in range(8):
                w = w_s[pl.ds(c * _C + s, 16)]
                iv = idx_s[4 * c + jr, pl.ds(jc, 16)]
                v = plsc.load_gather(tbl01_s, [iv])
                f0 = plsc.bitcast(v << 16, jnp.float32)
                f1 = plsc.bitcast(v & jnp.int32(-65536), jnp.float32)
                a0 = a0 + w * f0
                a1 = a1 + w * f1
            pb = j // 8
            poff = (j % 8) * 16
            out_s[pl.ds(ga0 * 4096 + pb * 1024 + ra0 * 128 + poff, 16)] = a0
            out_s[pl.ds(ga1 * 4096 + pb * 1024 + ra1 * 128 + poff, 16)] = a1
        plsc.parallel_loop(0, _C // 16, unroll=2)(acc_body)

    def level_acc(l, w_s, fp_s):
        ga0, ra0 = (2 * l) // 8, (2 * l) % 8
        ga1, ra1 = (2 * l + 1) // 8, (2 * l + 1) % 8

        def acc_body(j):
            s = j * 16
            jr = j // 8
            jc = (j % 8) * 16
            a0 = jnp.zeros((16,), jnp.float32)
            a1 = jnp.zeros((16,), jnp.float32)
            for c in range(8):
                w = w_s[pl.ds(c * _C + s, 16)]
                v = fp_s[4 * c + jr, pl.ds(jc, 16)]
                f0 = plsc.bitcast(v << 16, jnp.float32)
                f1 = plsc.bitcast(v & jnp.int32(-65536), jnp.float32)
                a0 = a0 + w * f0
                a1 = a1 + w * f1
            pb = j // 8
            poff = (j % 8) * 16
            out_s[pl.ds(ga0 * 4096 + pb * 1024 + ra0 * 128 + poff, 16)] = a0
            out_s[pl.ds(ga1 * 4096 + pb * 1024 + ra1 * 128 + poff, 16)] = a1
        plsc.parallel_loop(0, _C // 16, unroll=2)(acc_body)

    pltpu.sync_copy(tp_hbm.at[pl.ds(0, _T01)], tbl01_s)

    def chunk_body(g, carry):
        base = wid * _PW + g * _C
        pltpu.sync_copy(x_hbm.at[pl.ds(base, _C)], x_s)
        pltpu.sync_copy(y_hbm.at[pl.ds(base, _C)], y_s)
        pltpu.sync_copy(z_hbm.at[pl.ds(base, _C)], z_s)

        for l in (2, 3, 4):
            idx_s, w_s, fp_s, sem_x = slots[(l - 2) % 3]
            level_idx(l, idx_s, w_s)
            level_fire(idx_s, fp_s, sem_x)
        for l in (0, 1):
            level_idx(l, idx_d, w_d)
            level_acc_local(l, idx_d, w_d)
        for l in range(2, _NUM_LEVELS):
            idx_s, w_s, fp_s, sem_x = slots[(l - 2) % 3]
            level_drain(idx_s, fp_s, sem_x)
            level_acc(l, w_s, fp_s)
            if l + 3 < _NUM_LEVELS:
                level_idx(l + 3, idx_s, w_s)
                level_fire(idx_s, fp_s, sem_x)

        pbg = base // 128
        for fg in range(4):
            for pb in range(4):
                pltpu.async_copy(
                    out_s.at[pl.ds(fg * 4096 + pb * 1024, 1024)],
                    out_hbm.at[pl.ds((fg * _NPB + pbg + pb) * 1024, 1024)],
                    sem)
        for fg in range(4):
            for pb in range(4):
                pltpu.make_async_copy(
                    out_s.at[pl.ds(fg * 4096 + pb * 1024, 1024)],
                    out_hbm.at[pl.ds((fg * _NPB + pbg + pb) * 1024, 1024)],
                    sem).wait()
        return carry

    lax.fori_loop(0, _NCH, chunk_body, 0)


_hash_call = functools.partial(
    pl.kernel,
    mesh=plsc.VectorSubcoreMesh(core_axis_name="c", subcore_axis_name="s"),
    compiler_params=pltpu.CompilerParams(
        needs_layout_passes=False, use_tc_tiling_on_sc=False),
    out_type=jax.ShapeDtypeStruct((_NG * _NPB * 8 * 128,), jnp.float32),
    scratch_types=[
        pltpu.VMEM((_C,), jnp.float32),
        pltpu.VMEM((_C,), jnp.float32),
        pltpu.VMEM((_C,), jnp.float32),
        pltpu.VMEM((_NB, 128), jnp.int32),
        pltpu.VMEM((_NB, 128), jnp.int32),
        pltpu.VMEM((_NB, 128), jnp.int32),
        pltpu.VMEM((_NB, 128), jnp.int32),
        pltpu.VMEM((8 * _C,), jnp.float32),
        pltpu.VMEM((8 * _C,), jnp.float32),
        pltpu.VMEM((8 * _C,), jnp.float32),
        pltpu.VMEM((8 * _C,), jnp.float32),
        pltpu.VMEM((_NB, 128), jnp.int32),
        pltpu.VMEM((_NB, 128), jnp.int32),
        pltpu.VMEM((_NB, 128), jnp.int32),
        pltpu.VMEM((4 * 4 * 8 * 128,), jnp.float32),
        pltpu.VMEM((_T01,), jnp.int32),
        pltpu.SemaphoreType.DMA,
        pltpu.SemaphoreType.DMA,
        pltpu.SemaphoreType.DMA,
        pltpu.SemaphoreType.DMA,
    ],
)(_hash_body)


_PBG = 32


_PBG = 32


def _pe_body(x_ref, y_ref, z_ref, alias_ref, out_ref,
             sx_ref, sy_ref, sz_ref, cx_ref, cy_ref, cz_ref):
    del alias_ref
    g = pl.program_id(1)
    coords = (x_ref[...], y_ref[...], z_ref[...])

    def emit(rows):
        full = jnp.stack(rows, axis=0)
        out_ref[0] = jnp.transpose(full, (1, 0, 2))

    def load_state():
        return ((sx_ref[...], sy_ref[...], sz_ref[...]),
                (cx_ref[...], cy_ref[...], cz_ref[...]))

    def store_state(sin3, cos3):
        sx_ref[...], sy_ref[...], sz_ref[...] = sin3
        cx_ref[...], cy_ref[...], cz_ref[...] = cos3

    def double(sin3, cos3):
        s2 = tuple(2.0 * sv * cv for sv, cv in zip(sin3, cos3))
        c2 = tuple(1.0 - 2.0 * sv * sv for sv in sin3)
        return s2, c2

    @pl.when(g == 0)
    def _():
        sin3 = tuple(jnp.sin(c) for c in coords)
        cos3 = tuple(jnp.cos(c) for c in coords)
        store_state(sin3, cos3)
        emit([coords[0], coords[1], coords[2],
              sin3[0], sin3[1], sin3[2], cos3[0], cos3[1]])

    @pl.when(g == 1)
    def _():
        s0, c0 = load_state()
        s1, c1 = double(s0, c0)
        s2, c2 = double(s1, c1)
        store_state(s2, c2)
        emit([c0[2], s1[0], s1[1], s1[2], c1[0], c1[1], c1[2], s2[0]])

    @pl.when(g == 2)
    def _():
        s2, c2 = load_state()
        s3, c3 = double(s2, c2)
        store_state(s3, c3)
        emit([s2[1], s2[2], c2[0], c2[1], c2[2], s3[0], s3[1], s3[2]])

    @pl.when(g == 3)
    def _():
        s3, c3 = load_state()
        s4, c4 = double(s3, c3)
        store_state(s4, c4)
        emit([c3[0], c3[1], c3[2], s4[0], s4[1], s4[2], c4[0], c4[1]])

    @pl.when(g == 4)
    def _():
        s4, c4 = load_state()
        s5, c5 = double(s4, c4)
        emit([c4[2], s5[0], s5[1], s5[2], c5[0], c5[1], c5[2],
              jnp.zeros_like(coords[0])])


_pe_grid = (_NPB // _PBG, 5)

_pe_call = pl.pallas_call(
    _pe_body,
    grid=_pe_grid,
    in_specs=[
        pl.BlockSpec((_PBG, 128), lambda i, g: (i, 0)),
        pl.BlockSpec((_PBG, 128), lambda i, g: (i, 0)),
        pl.BlockSpec((_PBG, 128), lambda i, g: (i, 0)),
        pl.BlockSpec(memory_space=pl.ANY),
    ],
    out_specs=pl.BlockSpec((1, _PBG, 8, 128), lambda i, g: (4 + g, i, 0, 0)),
    out_shape=jax.ShapeDtypeStruct((_NG, _NPB, 8, 128), jnp.float32),
    input_output_aliases={3: 0},
    scratch_shapes=[pltpu.VMEM((_PBG, 128), jnp.float32) for _ in range(6)],
)


def kernel(position, table):
    xs = position[:, 0]
    ys = position[:, 1]
    zs = position[:, 2]
    b0 = jax.lax.bitcast_convert_type(
        table[:, 0].astype(jnp.bfloat16), jnp.uint16).astype(jnp.int32)
    b1 = jax.lax.bitcast_convert_type(
        table[:, 1].astype(jnp.bfloat16), jnp.uint16).astype(jnp.int32)
    tp = b0 | (b1 << 16)
    flat = _hash_call(xs, ys, zs, tp)
    x2 = xs.reshape(_NPB, 128)
    y2 = ys.reshape(_NPB, 128)
    z2 = zs.reshape(_NPB, 128)
    out4 = _pe_call(x2, y2, z2, flat.reshape(_NG, _NPB, 8, 128))
    return out4.transpose(1, 3, 0, 2).reshape(_N, _NG * 8)[:, :_OUT_DIM]


# speedup vs baseline: 1.2888x; 1.0740x over previous
import functools

import numpy as np
import jax
import jax.numpy as jnp
from jax import lax
from jax.experimental import pallas as pl
from jax.experimental.pallas import tpu as pltpu
from jax.experimental.pallas import tpu_sc as plsc

_NUM_LEVELS = 16
_BASE_RES = 16
_PER_LEVEL_SCALE = 2.0
_LOG2_HASHMAP = 19
_NUM_FREQS = 6
_N = 262144
_OUT_DIM = 2 * _NUM_LEVELS + 3 * (1 + 2 * _NUM_FREQS)
_HASH_DIM = 2 * _NUM_LEVELS
_NG = 9
_NPB = _N // 128


def _level_meta():
    hashmap = 2 ** _LOG2_HASHMAP
    offsets = [0]
    resolutions = []
    for l in range(_NUM_LEVELS):
        res = int(np.ceil(_BASE_RES * (_PER_LEVEL_SCALE ** l)))
        resolutions.append(res)
        params = min(hashmap, (res + 1) ** 3)
        params = int(np.ceil(params / 8) * 8)
        offsets.append(offsets[-1] + params)
    return offsets, resolutions


_OFFSETS, _RES = _level_meta()
_P1 = int(np.uint32(2654435761).astype(np.int64) - 2 ** 32)
_P2 = 805459861
_MASK = 2 ** _LOG2_HASHMAP - 1

_NW = 32
_PW = _N // _NW
_C = 512
_NCH = _PW // _C
_NB = 8 * _C // 128
_T01 = _OFFSETS[2]


def _hash_body(x_hbm, y_hbm, z_hbm, tp_hbm, out_hbm,
               x_s, y_s, z_s, idx_a, idx_b, idx_c, idx_d, w_a, w_b, w_c, w_d,
               fp_a, fp_b, fp_c, out_s, tbl01_s, sem, sem_a, sem_b, sem_c):
    wid = lax.axis_index("s") * 2 + lax.axis_index("c")
    iota = jnp.arange(16, dtype=jnp.int32)
    slots = ((idx_a, w_a, fp_a, sem_a), (idx_b, w_b, fp_b, sem_b),
             (idx_c, w_c, fp_c, sem_c))

    def level_idx(l, idx_s, w_s):
        res = _RES[l]
        off = _OFFSETS[l]
        n_params = _OFFSETS[l + 1] - _OFFSETS[l]
        hashed = (res + 1) ** 3 > n_params
        res_f = float(res)

        def idx_body(j):
            s = j * 16
            xf = x_s[pl.ds(s, 16)] * res_f
            yf = y_s[pl.ds(s, 16)] * res_f
            zf = z_s[pl.ds(s, 16)] * res_f
            xi = xf.astype(jnp.int32)
            yi = yf.astype(jnp.int32)
            zi = zf.astype(jnp.int32)
            fx = xf - xi.astype(jnp.float32)
            fy = yf - yi.astype(jnp.float32)
            fz = zf - zi.astype(jnp.float32)
            gx = 1.0 - fx
            gy = 1.0 - fy
            gz = 1.0 - fz
            if hashed:
                hy0 = yi * _P1
                hy1 = hy0 + _P1
                hz0 = zi * _P2
                hz1 = hz0 + _P2
            else:
                r1 = res + 1
                sy0 = yi * r1
                sy1 = sy0 + r1
                sz0 = zi * (r1 * r1)
                sz1 = sz0 + r1 * r1
            jr = j // 8
            jc = (j % 8) * 16
            for c in range(8):
                bx, by, bz = c & 1, (c >> 1) & 1, (c >> 2) & 1
                if hashed:
                    h = (xi + bx) ^ (hy1 if by else hy0) ^ (hz1 if bz else hz0)
                    idx = (h & _MASK) + off
                else:
                    idx = ((xi + bx) + (sy1 if by else sy0)
                           + (sz1 if bz else sz0) + off)
                w = ((fx if bx else gx) * (fy if by else gy)) * (fz if bz else gz)
                idx_s[4 * c + jr, pl.ds(jc, 16)] = idx
                w_s[pl.ds(c * _C + s, 16)] = w
        plsc.parallel_loop(0, _C // 16, unroll=2)(idx_body)

    def level_fire(idx_s, fp_s, sem_x):
        def fire_body(j):
            pltpu.async_copy(tp_hbm.at[idx_s.at[j]], fp_s.at[j], sem_x)
        plsc.parallel_loop(0, _NB, unroll=2)(fire_body)

    def level_drain(idx_s, fp_s, sem_x):
        def drain_body(j, c2):
            pltpu.make_async_copy(
                tp_hbm.at[idx_s.at[j]], fp_s.at[j], sem_x).wait()
            return c2
        lax.fori_loop(0, _NB, drain_body, 0)

    def level_acc_local(l, idx_s, w_s):
        ga0, ra0 = (2 * l) // 8, (2 * l) % 8
        ga1, ra1 = (2 * l + 1) // 8, (2 * l + 1) % 8

        def acc_body(j):
            s = j * 16
            jr = j // 8
            jc = (j % 8) * 16
            a0 = jnp.zeros((16,), jnp.float32)
            a1 = jnp.zeros((16,), jnp.float32)
            for c in range(8):
                w = w_s[pl.ds(c * _C + s, 16)]
                iv = idx_s[4 * c + jr, pl.ds(jc, 16)]
                v = plsc.load_gather(tbl01_s, [iv])
                f0 = plsc.bitcast(v << 16, jnp.float32)
                f1 = plsc.bitcast(v & jnp.int32(-65536), jnp.float32)
                a0 = a0 + w * f0
                a1 = a1 + w * f1
            pb = j // 8
            poff = (j % 8) * 16
            out_s[pl.ds(ga0 * 4096 + pb * 1024 + ra0 * 128 + poff, 16)] = a0
            out_s[pl.ds(ga1 * 4096 + pb * 1024 + ra1 * 128 + poff, 16)] = a1
        plsc.parallel_loop(0, _C // 16, unroll=2)(acc_body)

    def level_acc(l, w_s, fp_s):
        ga0, ra0 = (2 * l) // 8, (2 * l) % 8
        ga1, ra1 = (2 * l + 1) // 8, (2 * l + 1) % 8

        def acc_body(j):
            s = j * 16
            jr = j // 8
            jc = (j % 8) * 16
            a0 = jnp.zeros((16,), jnp.float32)
            a1 = jnp.zeros((16,), jnp.float32)
            for c in range(8):
                w = w_s[pl.ds(c * _C + s, 16)]
                v = fp_s[4 * c + jr, pl.ds(jc, 16)]
                f0 = plsc.bitcast(v << 16, jnp.float32)
                f1 = plsc.bitcast(v & jnp.int32(-65536), jnp.float32)
                a0 = a0 + w * f0
                a1 = a1 + w * f1
            pb = j // 8
            poff = (j % 8) * 16
            out_s[pl.ds(ga0 * 4096 + pb * 1024 + ra0 * 128 + poff, 16)] = a0
            out_s[pl.ds(ga1 * 4096 + pb * 1024 + ra1 * 128 + poff, 16)] = a1
        plsc.parallel_loop(0, _C // 16, unroll=2)(acc_body)

    pltpu.sync_copy(tp_hbm.at[pl.ds(0, _T01)], tbl01_s)

    def chunk_body(g, carry):
        base = wid * _PW + g * _C
        pltpu.sync_copy(x_hbm.at[pl.ds(base, _C)], x_s)
        pltpu.sync_copy(y_hbm.at[pl.ds(base, _C)], y_s)
        pltpu.sync_copy(z_hbm.at[pl.ds(base, _C)], z_s)

        for l in (2, 3, 4):
            idx_s, w_s, fp_s, sem_x = slots[(l - 2) % 3]
            level_idx(l, idx_s, w_s)
            level_fire(idx_s, fp_s, sem_x)
        for l in (0, 1):
            level_idx(l, idx_d, w_d)
            level_acc_local(l, idx_d, w_d)
        for l in range(2, _NUM_LEVELS):
            idx_s, w_s, fp_s, sem_x = slots[(l - 2) % 3]
            level_drain(idx_s, fp_s, sem_x)
            level_acc(l, w_s, fp_s)
            if l + 3 < _NUM_LEVELS:
                level_idx(l + 3, idx_s, w_s)
                level_fire(idx_s, fp_s, sem_x)

        pbg = base // 128
        for fg in range(4):
            for pb in range(4):
                pltpu.async_copy(
                    out_s.at[pl.ds(fg * 4096 + pb * 1024, 1024)],
                    out_hbm.at[pl.ds((fg * _NPB + pbg + pb) * 1024, 1024)],
                    sem)
        for fg in range(4):
            for pb in range(4):
                pltpu.make_async_copy(
                    out_s.at[pl.ds(fg * 4096 + pb * 1024, 1024)],
                    out_hbm.at[pl.ds((fg * _NPB + pbg + pb) * 1024, 1024)],
                    sem).wait()
        return carry

    lax.fori_loop(0, _NCH, chunk_body, 0)


_hash_call = functools.partial(
    pl.kernel,
    mesh=plsc.VectorSubcoreMesh(core_axis_name="c", subcore_axis_name="s"),
    compiler_params=pltpu.CompilerParams(
        needs_layout_passes=False, use_tc_tiling_on_sc=False),
    out_type=jax.ShapeDtypeStruct((4 * _NPB * 8 * 128,), jnp.float32),
    scratch_types=[
        pltpu.VMEM((_C,), jnp.float32),
        pltpu.VMEM((_C,), jnp.float32),
        pltpu.VMEM((_C,), jnp.float32),
        pltpu.VMEM((_NB, 128), jnp.int32),
        pltpu.VMEM((_NB, 128), jnp.int32),
        pltpu.VMEM((_NB, 128), jnp.int32),
        pltpu.VMEM((_NB, 128), jnp.int32),
        pltpu.VMEM((8 * _C,), jnp.float32),
        pltpu.VMEM((8 * _C,), jnp.float32),
        pltpu.VMEM((8 * _C,), jnp.float32),
        pltpu.VMEM((8 * _C,), jnp.float32),
        pltpu.VMEM((_NB, 128), jnp.int32),
        pltpu.VMEM((_NB, 128), jnp.int32),
        pltpu.VMEM((_NB, 128), jnp.int32),
        pltpu.VMEM((4 * 4 * 8 * 128,), jnp.float32),
        pltpu.VMEM((_T01,), jnp.int32),
        pltpu.SemaphoreType.DMA,
        pltpu.SemaphoreType.DMA,
        pltpu.SemaphoreType.DMA,
        pltpu.SemaphoreType.DMA,
    ],
)(_hash_body)


_PBG = 32


_PBG = 32


def _pe_body(x_ref, y_ref, z_ref, out_ref,
             sx_ref, sy_ref, sz_ref, cx_ref, cy_ref, cz_ref):
    g = pl.program_id(1)
    coords = (x_ref[...], y_ref[...], z_ref[...])

    def emit(rows):
        full = jnp.stack(rows, axis=0)
        out_ref[0] = jnp.transpose(full, (1, 0, 2))

    def load_state():
        return ((sx_ref[...], sy_ref[...], sz_ref[...]),
                (cx_ref[...], cy_ref[...], cz_ref[...]))

    def store_state(sin3, cos3):
        sx_ref[...], sy_ref[...], sz_ref[...] = sin3
        cx_ref[...], cy_ref[...], cz_ref[...] = cos3

    def double(sin3, cos3):
        s2 = tuple(2.0 * sv * cv for sv, cv in zip(sin3, cos3))
        c2 = tuple(1.0 - 2.0 * sv * sv for sv in sin3)
        return s2, c2

    @pl.when(g == 0)
    def _():
        sin3 = tuple(jnp.sin(c) for c in coords)
        cos3 = tuple(jnp.cos(c) for c in coords)
        store_state(sin3, cos3)
        emit([coords[0], coords[1], coords[2],
              sin3[0], sin3[1], sin3[2], cos3[0], cos3[1]])

    @pl.when(g == 1)
    def _():
        s0, c0 = load_state()
        s1, c1 = double(s0, c0)
        s2, c2 = double(s1, c1)
        store_state(s2, c2)
        emit([c0[2], s1[0], s1[1], s1[2], c1[0], c1[1], c1[2], s2[0]])

    @pl.when(g == 2)
    def _():
        s2, c2 = load_state()
        s3, c3 = double(s2, c2)
        store_state(s3, c3)
        emit([s2[1], s2[2], c2[0], c2[1], c2[2], s3[0], s3[1], s3[2]])

    @pl.when(g == 3)
    def _():
        s3, c3 = load_state()
        s4, c4 = double(s3, c3)
        store_state(s4, c4)
        emit([c3[0], c3[1], c3[2], s4[0], s4[1], s4[2], c4[0], c4[1]])

    @pl.when(g == 4)
    def _():
        s4, c4 = load_state()
        s5, c5 = double(s4, c4)
        emit([c4[2], s5[0], s5[1], s5[2], c5[0], c5[1], c5[2],
              jnp.zeros_like(coords[0])])


_pe_grid = (_NPB // _PBG, 5)

_pe_call = pl.pallas_call(
    _pe_body,
    grid=_pe_grid,
    in_specs=[
        pl.BlockSpec((_PBG, 128), lambda i, g: (i, 0)),
        pl.BlockSpec((_PBG, 128), lambda i, g: (i, 0)),
        pl.BlockSpec((_PBG, 128), lambda i, g: (i, 0)),
    ],
    out_specs=pl.BlockSpec((1, _PBG, 8, 128), lambda i, g: (g, i, 0, 0)),
    out_shape=jax.ShapeDtypeStruct((5, _NPB, 8, 128), jnp.float32),
    scratch_shapes=[pltpu.VMEM((_PBG, 128), jnp.float32) for _ in range(6)],
)


def kernel(position, table):
    xs = position[:, 0]
    ys = position[:, 1]
    zs = position[:, 2]
    b0 = jax.lax.bitcast_convert_type(
        table[:, 0].astype(jnp.bfloat16), jnp.uint16).astype(jnp.int32)
    b1 = jax.lax.bitcast_convert_type(
        table[:, 1].astype(jnp.bfloat16), jnp.uint16).astype(jnp.int32)
    tp = b0 | (b1 << 16)
    flat = _hash_call(xs, ys, zs, tp)
    x2 = xs.reshape(_NPB, 128)
    y2 = ys.reshape(_NPB, 128)
    z2 = zs.reshape(_NPB, 128)
    pe5 = _pe_call(x2, y2, z2)
    out4 = jnp.concatenate([flat.reshape(4, _NPB, 8, 128), pe5], axis=0)
    return out4.transpose(1, 3, 0, 2).reshape(_N, _NG * 8)[:, :_OUT_DIM]
